# baseline (speedup 1.0000x reference)
import jax
import jax.numpy as jnp
from jax.experimental import pallas as pl
from jax.experimental.pallas import tpu as pltpu

_B = 400


def _body(x_ref, t_ref, w_ref, b_ref, o_ref):
    xb = x_ref[...].astype(jnp.bfloat16)
    tb = t_ref[...]
    H = o_ref.shape[1]
    T = w_ref.shape[1] // H
    p = jnp.dot(xb, w_ref[...], preferred_element_type=jnp.float32)
    acc = p[:, 0:H] + b_ref[0][None, :]
    for t in range(1, T):
        acc = jnp.where(tb == t, p[:, t * H:(t + 1) * H] + b_ref[t][None, :], acc)
    o_ref[...] = acc


def kernel(x, node_types, W, b):
    N, D = x.shape
    T, H, _ = W.shape
    assert N % _B == 0
    nt2 = node_types.astype(jnp.int32).reshape(N, 1)
    Wc = jnp.swapaxes(W, 1, 2).transpose(1, 0, 2).reshape(D, T * H).astype(jnp.bfloat16)
    return pl.pallas_call(
        _body,
        grid=(N // _B,),
        in_specs=[
            pl.BlockSpec((_B, D), lambda i: (i, 0)),
            pl.BlockSpec((_B, 1), lambda i: (i, 0)),
            pl.BlockSpec((D, T * H), lambda i: (0, 0)),
            pl.BlockSpec((T, H), lambda i: (0, 0)),
        ],
        out_specs=pl.BlockSpec((_B, H), lambda i: (i, 0)),
        out_shape=jax.ShapeDtypeStruct((N, H), x.dtype),
        compiler_params=pltpu.CompilerParams(
            dimension_semantics=("parallel",),
        ),
    )(x, nt2, Wc, b)


# speedup vs baseline: 1.7190x; 1.7190x over previous
import jax
import jax.numpy as jnp
from jax import lax
from jax.experimental import pallas as pl
from jax.experimental.pallas import tpu as pltpu

_B = 2000


def _outer(x_hbm, t_hbm, w_ref, b_ref, o_hbm, xbuf, tbuf, obuf, sx, st, so):
    i = pl.program_id(0)
    G = pl.num_programs(0)
    slot = lax.rem(i, 2)
    nslot = lax.rem(i + 1, 2)
    H = b_ref.shape[1]
    T = b_ref.shape[0]

    def in_copies(blk, sl):
        return (
            pltpu.make_async_copy(
                x_hbm.at[pl.ds(blk * _B, _B), :], xbuf.at[sl], sx.at[sl]),
            pltpu.make_async_copy(
                t_hbm.at[pl.ds(blk * _B, _B), :], tbuf.at[sl], st.at[sl]),
        )

    def out_copy(blk, sl):
        return pltpu.make_async_copy(
            obuf.at[sl], o_hbm.at[pl.ds(blk * _B, _B), :], so.at[sl])

    @pl.when(i == 0)
    def _():
        c1, c2 = in_copies(0, 0)
        c1.start()
        c2.start()

    @pl.when(i + 1 < G)
    def _():
        c1, c2 = in_copies(i + 1, nslot)
        c1.start()
        c2.start()

    c1, c2 = in_copies(i, slot)
    c1.wait()
    c2.wait()

    @pl.when(i >= 2)
    def _():
        out_copy(i - 2, slot).wait()

    xb = xbuf[slot].astype(jnp.bfloat16)
    tb = tbuf[slot]
    p = jnp.dot(xb, w_ref[...], preferred_element_type=jnp.float32)
    acc = p[:, 0:H] + b_ref[0][None, :]
    for t in range(1, T):
        acc = jnp.where(tb == t, p[:, t * H:(t + 1) * H] + b_ref[t][None, :], acc)
    obuf[slot] = acc

    out_copy(i, slot).start()

    @pl.when(i == G - 1)
    def _():
        @pl.when(G >= 2)
        def _():
            out_copy(G - 2, nslot).wait()
        out_copy(G - 1, slot).wait()


def kernel(x, node_types, W, b):
    N, D = x.shape
    T, H, _ = W.shape
    assert N % _B == 0
    G = N // _B
    nt2 = node_types.astype(jnp.int32).reshape(N, 1)
    Wc = jnp.swapaxes(W, 1, 2).transpose(1, 0, 2).reshape(D, T * H).astype(jnp.bfloat16)
    return pl.pallas_call(
        _outer,
        grid=(G,),
        in_specs=[
            pl.BlockSpec(memory_space=pltpu.HBM),
            pl.BlockSpec(memory_space=pltpu.HBM),
            pl.BlockSpec(memory_space=pltpu.VMEM),
            pl.BlockSpec(memory_space=pltpu.VMEM),
        ],
        out_specs=pl.BlockSpec(memory_space=pltpu.HBM),
        out_shape=jax.ShapeDtypeStruct((N, H), x.dtype),
        scratch_shapes=[
            pltpu.VMEM((2, _B, D), jnp.float32),
            pltpu.VMEM((2, _B, 1), jnp.int32),
            pltpu.VMEM((2, _B, H), jnp.float32),
            pltpu.SemaphoreType.DMA((2,)),
            pltpu.SemaphoreType.DMA((2,)),
            pltpu.SemaphoreType.DMA((2,)),
        ],
        compiler_params=pltpu.CompilerParams(
            dimension_semantics=("arbitrary",),
        ),
    )(x, nt2, Wc, b)


# speedup vs baseline: 2.0956x; 1.2191x over previous
import jax
import jax.numpy as jnp
from jax import lax
from jax.experimental import pallas as pl
from jax.experimental.pallas import tpu as pltpu

_B = 2000
_DEPTH = 4


def _outer(x_hbm, t_hbm, w_ref, b_ref, o_hbm, xbuf, tbuf, obuf, sx, st, so):
    i = pl.program_id(0)
    G = pl.num_programs(0)
    slot = lax.rem(i, _DEPTH)
    H = b_ref.shape[1]
    T = b_ref.shape[0]

    def in_copies(blk, sl):
        return (
            pltpu.make_async_copy(
                x_hbm.at[pl.ds(blk * _B, _B), :], xbuf.at[sl], sx.at[sl]),
            pltpu.make_async_copy(
                t_hbm.at[pl.ds(blk * _B, _B), :], tbuf.at[sl], st.at[sl]),
        )

    def out_copy(blk, sl):
        return pltpu.make_async_copy(
            obuf.at[sl], o_hbm.at[pl.ds(blk * _B, _B), :], so.at[sl])

    @pl.when(i == 0)
    def _():
        for k in range(min(_DEPTH, G)):
            c1, c2 = in_copies(k, k)
            c1.start()
            c2.start()

    c1, c2 = in_copies(i, slot)
    c1.wait()
    c2.wait()

    @pl.when(i >= _DEPTH)
    def _():
        out_copy(i - _DEPTH, slot).wait()

    xb = xbuf[slot].astype(jnp.bfloat16)
    tb = tbuf[slot]
    p = jnp.dot(xb, w_ref[...], preferred_element_type=jnp.float32)
    acc = p[:, 0:H] + b_ref[0][None, :]
    for t in range(1, T):
        acc = jnp.where(tb == t, p[:, t * H:(t + 1) * H] + b_ref[t][None, :], acc)
    obuf[slot] = acc

    out_copy(i, slot).start()

    @pl.when(i + _DEPTH < G)
    def _():
        c1, c2 = in_copies(i + _DEPTH, slot)
        c1.start()
        c2.start()

    @pl.when(i == G - 1)
    def _():
        for k in range(min(_DEPTH - 1, G - 1)):
            blk = G - 1 - (k + 1)
            out_copy(blk, lax.rem(blk, _DEPTH)).wait()
        out_copy(G - 1, slot).wait()


def kernel(x, node_types, W, b):
    N, D = x.shape
    T, H, _ = W.shape
    assert N % _B == 0
    G = N // _B
    nt2 = node_types.astype(jnp.int32).reshape(N, 1)
    Wc = jnp.swapaxes(W, 1, 2).transpose(1, 0, 2).reshape(D, T * H).astype(jnp.bfloat16)
    return pl.pallas_call(
        _outer,
        grid=(G,),
        in_specs=[
            pl.BlockSpec(memory_space=pltpu.HBM),
            pl.BlockSpec(memory_space=pltpu.HBM),
            pl.BlockSpec(memory_space=pltpu.VMEM),
            pl.BlockSpec(memory_space=pltpu.VMEM),
        ],
        out_specs=pl.BlockSpec(memory_space=pltpu.HBM),
        out_shape=jax.ShapeDtypeStruct((N, H), x.dtype),
        scratch_shapes=[
            pltpu.VMEM((_DEPTH, _B, D), jnp.float32),
            pltpu.VMEM((_DEPTH, _B, 1), jnp.int32),
            pltpu.VMEM((_DEPTH, _B, H), jnp.float32),
            pltpu.SemaphoreType.DMA((_DEPTH,)),
            pltpu.SemaphoreType.DMA((_DEPTH,)),
            pltpu.SemaphoreType.DMA((_DEPTH,)),
        ],
        compiler_params=pltpu.CompilerParams(
            dimension_semantics=("arbitrary",),
        ),
    )(x, nt2, Wc, b)
